# baseline (speedup 1.0000x reference)
import math

import jax
import jax.numpy as jnp
import numpy as np
from jax.experimental import pallas as pl

N = 10000
E = 320000
NUM_BASIS = 10
RADIAL_NEURONS = 128
MAX_RADIUS = 5.0
NUM_NEIGHBORS = 32.0


def _silu(x):
    return x * jax.nn.sigmoid(x)


def _smooth_cutoff(x):
    u = 2.0 * (x - 1.0)
    y = (1.0 - jnp.cos(jnp.pi * u)) / 2.0
    y = jnp.where(u > 0.0, 0.0, y)
    y = jnp.where(u < -1.0, 1.0, y)
    return y


def _soft_one_hot(x, start, end, number):
    values = jnp.linspace(start, end, number)
    step = values[1] - values[0]
    diff = (x[:, None] - values[None, :]) / step
    return jnp.exp(-diff ** 2) / 1.12


def _radial_mlp(emb, p):
    h = _silu(emb @ p['fc_w0'] / np.sqrt(NUM_BASIS))
    h = _silu(h @ p['fc_w1'] / np.sqrt(RADIAL_NEURONS))
    h = h @ p['fc_w2'] / np.sqrt(RADIAL_NEURONS)
    return h


def _conv(x, p, edge_src, edge_dst, edge_emb, edge_cutoff, n_nodes):
    d_in = x.shape[1]
    w = _radial_mlp(edge_emb, p)
    x_sc = x @ p['W_sc'] / np.sqrt(d_in)
    xl = x @ p['W_lin1'] / np.sqrt(d_in)
    ef = xl[edge_src] * w * edge_cutoff[:, None]
    agg = jax.ops.segment_sum(ef, edge_dst, num_segments=n_nodes) / np.sqrt(NUM_NEIGHBORS)
    out = agg @ p['W_lin2'] / np.sqrt(d_in)
    c_s, c_x = math.sin(math.pi / 8), math.cos(math.pi / 8)
    return c_s * x_sc + c_x * out


def _network(x, pos, edge_src, edge_dst, layers):
    ev = pos[edge_src] - pos[edge_dst]
    el = jnp.sqrt(jnp.sum(ev ** 2, axis=1) + 1e-9)
    emb = _soft_one_hot(el, 0.0, MAX_RADIUS, NUM_BASIS) * (NUM_BASIS ** 0.5)
    cut = _smooth_cutoff(el / MAX_RADIUS)
    h = x
    n = x.shape[0]
    for i, lp in enumerate(layers):
        h = _conv(h, lp, edge_src, edge_dst, emb, cut, n)
        if i < len(layers) - 1:
            h = _silu(h)
    return h


def kernel(x, x_final_state, pos, pos_final_state, pos_interpolated_transition_state, p, edge_index, batch, params):
    edge_src = edge_index[0]
    edge_dst = edge_index[1]
    out_i = _network(x, pos, edge_src, edge_dst, params['net_init'])
    out_f = _network(x_final_state, pos_final_state, edge_src, edge_dst, params['net_final'])
    p0 = p[0]
    x_ts = p0 * out_i + (1.0 - p0) * out_f
    out_ts = _network(x_ts, pos_interpolated_transition_state, edge_src, edge_dst, params['net_ts'])
    return out_ts


# speedup vs baseline: 1.2652x; 1.2652x over previous
import dataclasses
import functools
import math

import jax
import jax.numpy as jnp
import numpy as np
from jax import lax
from jax.experimental import pallas as pl
from jax.experimental.pallas import tpu as pltpu
from jax.experimental.pallas import tpu_sc as plsc

N = 10000
E = 320000
EP = 327680
NUM_BASIS = 10
MAX_RADIUS = 5.0

_MESH = plsc.VectorSubcoreMesh(core_axis_name="c", subcore_axis_name="s")
_SC_PARAMS = pltpu.CompilerParams()
if "needs_layout_passes" in pltpu.CompilerParams.__dataclass_fields__:
    _SC_PARAMS = dataclasses.replace(_SC_PARAMS, needs_layout_passes=False)
_NS = 16
_EPT_G = E // 32
_ROWS_PT = 632
_NP = _ROWS_PT * _NS
_CHUNK = 80

_BE = 2048
_BN = 2000


def _silu(v):
    return v / (1.0 + jnp.exp(-v))


def _geom(px, py, pz, src, dst):
    @functools.partial(
        pl.kernel,
        out_type=jax.ShapeDtypeStruct((EP,), jnp.float32),
        mesh=_MESH,
        compiler_params=_SC_PARAMS,
        scratch_types=[
            pltpu.VMEM((N,), jnp.float32),
            pltpu.VMEM((N,), jnp.float32),
            pltpu.VMEM((N,), jnp.float32),
            pltpu.VMEM((_EPT_G,), jnp.int32),
            pltpu.VMEM((_EPT_G,), jnp.int32),
            pltpu.VMEM((_EPT_G,), jnp.float32),
        ],
    )
    def k(px_h, py_h, pz_h, src_h, dst_h, out_h, px_v, py_v, pz_v, si_v, di_v, o_v):
        c = lax.axis_index("c")
        s = lax.axis_index("s")
        base = (c * _NS + s) * _EPT_G
        pltpu.sync_copy(px_h, px_v)
        pltpu.sync_copy(py_h, py_v)
        pltpu.sync_copy(pz_h, pz_v)
        pltpu.sync_copy(src_h.at[pl.ds(base, _EPT_G)], si_v)
        pltpu.sync_copy(dst_h.at[pl.ds(base, _EPT_G)], di_v)

        @pl.loop(0, _EPT_G, step=16)
        def _(i):
            sl = pl.ds(i, 16)
            a = si_v[sl]
            b = di_v[sl]
            dx = plsc.load_gather(px_v, [a]) - plsc.load_gather(px_v, [b])
            dy = plsc.load_gather(py_v, [a]) - plsc.load_gather(py_v, [b])
            dz = plsc.load_gather(pz_v, [a]) - plsc.load_gather(pz_v, [b])
            o_v[sl] = dx * dx + dy * dy + dz * dz

        pltpu.sync_copy(o_v, out_h.at[pl.ds(base, _EPT_G)])

    return k(px, py, pz, src, dst)


def _sparse_layer(split, xl, we, src, dst):
    ept = E // _NS if split else E // (2 * _NS)
    nch = ept // _CHUNK

    @functools.partial(
        pl.kernel,
        out_type=jax.ShapeDtypeStruct((2 * _NP, 128), jnp.float32),
        mesh=_MESH,
        compiler_params=_SC_PARAMS,
        scratch_types=[
            pltpu.VMEM((_CHUNK,), jnp.int32),
            pltpu.VMEM((_CHUNK,), jnp.int32),
            pltpu.VMEM((_CHUNK, 128), jnp.float32),
            pltpu.VMEM((_CHUNK, 128), jnp.float32),
            pltpu.VMEM_SHARED((_NP, 128), jnp.float32),
        ],
    )
    def k(xl_h, we_h, src_h, dst_h, out_h, si_v, di_v, rows_v, we_v, agg_sh):
        c = lax.axis_index("c")
        s = lax.axis_index("s")

        @pl.loop(0, _CHUNK)
        def _(i):
            for jj in range(8):
                rows_v[i, pl.ds(jj * 16, 16)] = jnp.zeros((16,), jnp.float32)

        zbase = s * _ROWS_PT
        for t in range(7):
            pltpu.sync_copy(rows_v, agg_sh.at[pl.ds(zbase + t * _CHUNK, _CHUNK)])
        pltpu.sync_copy(rows_v.at[pl.ds(0, 72)], agg_sh.at[pl.ds(zbase + 560, 72)])
        plsc.subcore_barrier()

        if split:
            ebase0 = s * ept
        else:
            ebase0 = (c * _NS + s) * ept

        @pl.loop(0, nch)
        def _(j):
            base = ebase0 + j * _CHUNK
            pltpu.sync_copy(src_h.at[pl.ds(base, _CHUNK)], si_v)
            pltpu.sync_copy(dst_h.at[pl.ds(base, _CHUNK)], di_v)
            if split:
                off = c * N
                for k16 in range(_CHUNK // 16):
                    sl = pl.ds(k16 * 16, 16)
                    si_v[sl] = si_v[sl] + off
            pltpu.sync_copy(xl_h.at[si_v], rows_v)
            wb = (c * EP + base) if split else base
            pltpu.sync_copy(we_h.at[pl.ds(wb, _CHUNK)], we_v)

            @pl.loop(0, _CHUNK)
            def _(i):
                for jj in range(8):
                    sl = pl.ds(jj * 16, 16)
                    rows_v[i, sl] = rows_v[i, sl] * we_v[i, sl]

            pltpu.sync_copy(rows_v, agg_sh.at[di_v], add=True)

        plsc.subcore_barrier()
        pltpu.sync_copy(
            agg_sh.at[pl.ds(zbase, _ROWS_PT)],
            out_h.at[pl.ds(c * _NP + zbase, _ROWS_PT)],
        )

    return k(xl, we, src, dst)


def _radial(dl2, layers):
    d2 = dl2.reshape(EP, 1)
    vals = np.linspace(0.0, MAX_RADIUS, NUM_BASIS).astype(np.float32)
    step = float(vals[1] - vals[0])
    hs = [w["fc_w2h"].shape[0] for w in layers]

    def body(d2_ref, *refs):
        wrefs = refs[:9]
        orefs = refs[9:]
        el = jnp.sqrt(d2_ref[...] + 1e-9)
        vgrid = (
            lax.broadcasted_iota(jnp.int32, (1, NUM_BASIS), 1).astype(jnp.float32)
            * step
        )
        diff = (el - vgrid) * (1.0 / step)
        emb = jnp.exp(-diff * diff) * (math.sqrt(NUM_BASIS) / 1.12)
        u = el * (2.0 / MAX_RADIUS) - 2.0
        y = (1.0 - jnp.cos(np.float32(math.pi) * u)) * 0.5
        y = jnp.where(u > 0.0, 0.0, y)
        y = jnp.where(u < -1.0, 1.0, y)
        for li in range(3):
            f0, f1, f2 = wrefs[3 * li : 3 * li + 3]
            h0 = _silu(jnp.dot(emb, f0[...], preferred_element_type=jnp.float32, precision=lax.Precision.HIGHEST))
            h1 = _silu(jnp.dot(h0, f1[...], preferred_element_type=jnp.float32, precision=lax.Precision.HIGHEST))
            for hh in range(hs[li]):
                w = jnp.dot(h1, f2[hh], preferred_element_type=jnp.float32, precision=lax.Precision.HIGHEST)
                if hs[li] == 1:
                    orefs[li][...] = w * y
                else:
                    orefs[li][hh] = w * y

    in_specs = [pl.BlockSpec((_BE, 1), lambda i: (i, 0))]
    args = [d2]
    for w in layers:
        args += [w["fc_w0"], w["fc_w1"], w["fc_w2h"]]
        in_specs += [
            pl.BlockSpec((NUM_BASIS, 128), lambda i: (0, 0)),
            pl.BlockSpec((128, 128), lambda i: (0, 0)),
            pl.BlockSpec((w["fc_w2h"].shape[0], 128, 128), lambda i: (0, 0, 0)),
        ]
    out_shapes = []
    out_specs = []
    for h in hs:
        if h == 1:
            out_shapes.append(jax.ShapeDtypeStruct((EP, 128), jnp.float32))
            out_specs.append(pl.BlockSpec((_BE, 128), lambda i: (i, 0)))
        else:
            out_shapes.append(jax.ShapeDtypeStruct((h, EP, 128), jnp.float32))
            out_specs.append(pl.BlockSpec((h, _BE, 128), lambda i: (0, i, 0)))
    outs = pl.pallas_call(
        body,
        grid=(EP // _BE,),
        in_specs=in_specs,
        out_specs=out_specs,
        out_shape=out_shapes,
    )(*args)
    return [o.reshape(-1, 128) for o in outs]


def _pre(h, wsc, w1h, h2=None, p=None):
    d_in = h.shape[1]
    d_out = wsc.shape[1]
    H = w1h.shape[0]

    def body(*refs):
        if p is None:
            h_ref, wsc_ref, w1_ref, xs_ref, xl_ref = refs
            hb = h_ref[...]
        else:
            h_ref, h2_ref, p_ref, wsc_ref, w1_ref, xs_ref, xl_ref = refs
            pv = p_ref[0, 0]
            hb = pv * h_ref[...] + (1.0 - pv) * h2_ref[...]
        xs_ref[...] = jnp.dot(hb, wsc_ref[...], preferred_element_type=jnp.float32, precision=lax.Precision.HIGHEST)
        for k in range(H):
            xl_ref[k] = jnp.dot(hb, w1_ref[k], preferred_element_type=jnp.float32, precision=lax.Precision.HIGHEST)

    in_specs = [pl.BlockSpec((_BN, d_in), lambda i: (i, 0))]
    args = [h]
    if p is not None:
        in_specs += [
            pl.BlockSpec((_BN, d_in), lambda i: (i, 0)),
            pl.BlockSpec((1, 1), lambda i: (0, 0)),
        ]
        args += [h2, p.reshape(1, 1)]
    in_specs += [
        pl.BlockSpec((d_in, d_out), lambda i: (0, 0)),
        pl.BlockSpec((H, d_in, 128), lambda i: (0, 0, 0)),
    ]
    args += [wsc, w1h]
    xs, xl = pl.pallas_call(
        body,
        grid=(N // _BN,),
        in_specs=in_specs,
        out_specs=[
            pl.BlockSpec((_BN, d_out), lambda i: (i, 0)),
            pl.BlockSpec((H, _BN, 128), lambda i: (0, i, 0)),
        ],
        out_shape=[
            jax.ShapeDtypeStruct((N, d_out), jnp.float32),
            jax.ShapeDtypeStruct((H, N, 128), jnp.float32),
        ],
    )(*args)
    return xs, xl.reshape(H * N, 128)


def _combine(agg, xs, w2h, act):
    d_out = xs.shape[1]
    a = agg.reshape(2, _NP, 128)

    def body(a_ref, xs_ref, w2_ref, o_ref):
        o = (
            xs_ref[...]
            + jnp.dot(a_ref[0], w2_ref[0], preferred_element_type=jnp.float32, precision=lax.Precision.HIGHEST)
            + jnp.dot(a_ref[1], w2_ref[1], preferred_element_type=jnp.float32, precision=lax.Precision.HIGHEST)
        )
        if act:
            o = _silu(o)
        o_ref[...] = o

    return pl.pallas_call(
        body,
        grid=(N // _BN,),
        in_specs=[
            pl.BlockSpec((2, _BN, 128), lambda i: (0, i, 0)),
            pl.BlockSpec((_BN, d_out), lambda i: (i, 0)),
            pl.BlockSpec((2, 128, d_out), lambda i: (0, 0, 0)),
        ],
        out_specs=pl.BlockSpec((_BN, d_out), lambda i: (i, 0)),
        out_shape=jax.ShapeDtypeStruct((N, d_out), jnp.float32),
    )(a, xs, w2h)


def _prep_params(layers):
    c_s, c_x = math.sin(math.pi / 8), math.cos(math.pi / 8)
    out = []
    for lp in layers:
        d_in = lp["W_sc"].shape[0]
        H = d_in // 128
        w1 = (lp["W_lin1"] * (1.0 / math.sqrt(d_in))).reshape(d_in, H, 128)
        f2 = (
            lp["fc_w2"]
            * (c_x / (math.sqrt(128.0) * math.sqrt(32.0) * math.sqrt(d_in)))
        ).reshape(128, H, 128)
        w2 = lp["W_lin2"].reshape(H, 128, -1)
        if H == 1:
            w2 = jnp.concatenate([w2, w2], axis=0)
        out.append(
            {
                "W_sc": lp["W_sc"] * (c_s / math.sqrt(d_in)),
                "W_lin1h": jnp.transpose(w1, (1, 0, 2)),
                "fc_w0": lp["fc_w0"] * (1.0 / math.sqrt(NUM_BASIS)),
                "fc_w1": lp["fc_w1"] * (1.0 / math.sqrt(128.0)),
                "fc_w2h": jnp.transpose(f2, (1, 0, 2)),
                "W_lin2h": w2,
                "H": H,
            }
        )
    return out


def _network(h, pos, src, dst, layers, h2=None, p=None):
    px, py, pz = pos[:, 0], pos[:, 1], pos[:, 2]
    dl2 = _geom(px, py, pz, src, dst)
    wes = _radial(dl2, layers)
    for li, lp in enumerate(layers):
        if li == 0 and p is not None:
            xs, xl = _pre(h, lp["W_sc"], lp["W_lin1h"], h2=h2, p=p)
        else:
            xs, xl = _pre(h, lp["W_sc"], lp["W_lin1h"])
        agg = _sparse_layer(lp["H"] == 2, xl, wes[li], src, dst)
        h = _combine(agg, xs, lp["W_lin2h"], act=(li < 2))
    return h


def kernel(x, x_final_state, pos, pos_final_state, pos_interpolated_transition_state, p, edge_index, batch, params):
    src = edge_index[0].astype(jnp.int32)
    dst = edge_index[1].astype(jnp.int32)
    net_i = _prep_params(params["net_init"])
    net_f = _prep_params(params["net_final"])
    net_ts = _prep_params(params["net_ts"])
    out_i = _network(x, pos, src, dst, net_i)
    out_f = _network(x_final_state, pos_final_state, src, dst, net_f)
    out_ts = _network(
        out_i,
        pos_interpolated_transition_state,
        src,
        dst,
        net_ts,
        h2=out_f,
        p=p[0],
    )
    return out_ts


# speedup vs baseline: 1.7397x; 1.3751x over previous
import dataclasses
import functools
import math

import jax
import jax.numpy as jnp
import numpy as np
from jax import lax
from jax.experimental import pallas as pl
from jax.experimental.pallas import tpu as pltpu
from jax.experimental.pallas import tpu_sc as plsc

N = 10000
E = 320000
EP = 327680
NUM_BASIS = 10
MAX_RADIUS = 5.0

_MESH = plsc.VectorSubcoreMesh(core_axis_name="c", subcore_axis_name="s")
_SC_PARAMS = pltpu.CompilerParams()
if "needs_layout_passes" in pltpu.CompilerParams.__dataclass_fields__:
    _SC_PARAMS = dataclasses.replace(_SC_PARAMS, needs_layout_passes=False)
_NS = 16
_EPT_G = E // 32
_ROWS_PT = 632
_NP = _ROWS_PT * _NS
_CHUNK = 80

_BE = 2048
_BN = 2000


def _silu(v):
    return v * jax.nn.sigmoid(v)


def _geom(px, py, pz, src, dst):
    @functools.partial(
        pl.kernel,
        out_type=jax.ShapeDtypeStruct((EP,), jnp.float32),
        mesh=_MESH,
        compiler_params=_SC_PARAMS,
        scratch_types=[
            pltpu.VMEM((N,), jnp.float32),
            pltpu.VMEM((N,), jnp.float32),
            pltpu.VMEM((N,), jnp.float32),
            pltpu.VMEM((_EPT_G,), jnp.int32),
            pltpu.VMEM((_EPT_G,), jnp.int32),
            pltpu.VMEM((_EPT_G,), jnp.float32),
        ],
    )
    def k(px_h, py_h, pz_h, src_h, dst_h, out_h, px_v, py_v, pz_v, si_v, di_v, o_v):
        c = lax.axis_index("c")
        s = lax.axis_index("s")
        base = (c * _NS + s) * _EPT_G
        pltpu.sync_copy(px_h, px_v)
        pltpu.sync_copy(py_h, py_v)
        pltpu.sync_copy(pz_h, pz_v)
        pltpu.sync_copy(src_h.at[pl.ds(base, _EPT_G)], si_v)
        pltpu.sync_copy(dst_h.at[pl.ds(base, _EPT_G)], di_v)

        @pl.loop(0, _EPT_G, step=16)
        def _(i):
            sl = pl.ds(i, 16)
            a = si_v[sl]
            b = di_v[sl]
            dx = plsc.load_gather(px_v, [a]) - plsc.load_gather(px_v, [b])
            dy = plsc.load_gather(py_v, [a]) - plsc.load_gather(py_v, [b])
            dz = plsc.load_gather(pz_v, [a]) - plsc.load_gather(pz_v, [b])
            o_v[sl] = dx * dx + dy * dy + dz * dz

        pltpu.sync_copy(o_v, out_h.at[pl.ds(base, _EPT_G)])

    return k(px, py, pz, src, dst)


def _sparse_layer(split, xl, we, src, dst):
    ept = E // _NS if split else E // (2 * _NS)
    nch = ept // _CHUNK

    @functools.partial(
        pl.kernel,
        out_type=jax.ShapeDtypeStruct((2 * _NP, 128), jnp.float32),
        mesh=_MESH,
        compiler_params=_SC_PARAMS,
        scratch_types=[
            pltpu.VMEM((_CHUNK,), jnp.int32),
            pltpu.VMEM((_CHUNK,), jnp.int32),
            pltpu.VMEM((_CHUNK, 128), jnp.float32),
            pltpu.VMEM((_CHUNK, 128), jnp.float32),
            pltpu.VMEM_SHARED((_NP, 128), jnp.float32),
        ],
    )
    def k(xl_h, we_h, src_h, dst_h, out_h, si_v, di_v, rows_v, we_v, agg_sh):
        c = lax.axis_index("c")
        s = lax.axis_index("s")

        @pl.loop(0, _CHUNK)
        def _(i):
            for jj in range(8):
                rows_v[i, pl.ds(jj * 16, 16)] = jnp.zeros((16,), jnp.float32)

        zbase = s * _ROWS_PT
        for t in range(7):
            pltpu.sync_copy(rows_v, agg_sh.at[pl.ds(zbase + t * _CHUNK, _CHUNK)])
        pltpu.sync_copy(rows_v.at[pl.ds(0, 72)], agg_sh.at[pl.ds(zbase + 560, 72)])
        plsc.subcore_barrier()

        if split:
            ebase0 = s * ept
        else:
            ebase0 = (c * _NS + s) * ept

        @pl.loop(0, nch)
        def _(j):
            base = ebase0 + j * _CHUNK
            pltpu.sync_copy(src_h.at[pl.ds(base, _CHUNK)], si_v)
            pltpu.sync_copy(dst_h.at[pl.ds(base, _CHUNK)], di_v)
            if split:
                off = c * N
                for k16 in range(_CHUNK // 16):
                    sl = pl.ds(k16 * 16, 16)
                    si_v[sl] = si_v[sl] + off
            pltpu.sync_copy(xl_h.at[si_v], rows_v)
            wb = (c * EP + base) if split else base
            pltpu.sync_copy(we_h.at[pl.ds(wb, _CHUNK)], we_v)

            @pl.loop(0, _CHUNK)
            def _(i):
                for jj in range(8):
                    sl = pl.ds(jj * 16, 16)
                    rows_v[i, sl] = rows_v[i, sl] * we_v[i, sl]

            pltpu.sync_copy(rows_v, agg_sh.at[di_v], add=True)

        plsc.subcore_barrier()
        pltpu.sync_copy(
            agg_sh.at[pl.ds(zbase, _ROWS_PT)],
            out_h.at[pl.ds(c * _NP + zbase, _ROWS_PT)],
        )

    return k(xl, we, src, dst)


def _radial(dl2, layers):
    d2 = dl2.reshape(EP, 1)
    vals = np.linspace(0.0, MAX_RADIUS, NUM_BASIS).astype(np.float32)
    step = float(vals[1] - vals[0])
    hs = [w["fc_w2h"].shape[0] for w in layers]

    def body(d2_ref, *refs):
        wrefs = refs[:9]
        orefs = refs[9:]
        el = jnp.sqrt(d2_ref[...] + 1e-9)
        vgrid = (
            lax.broadcasted_iota(jnp.int32, (1, NUM_BASIS), 1).astype(jnp.float32)
            * step
        )
        diff = (el - vgrid) / step
        emb = (jnp.exp(-diff * diff) / 1.12) * (NUM_BASIS ** 0.5)
        u = 2.0 * (el / MAX_RADIUS - 1.0)
        y = (1.0 - jnp.cos(np.float32(math.pi) * u)) / 2.0
        y = jnp.where(u > 0.0, 0.0, y)
        y = jnp.where(u < -1.0, 1.0, y)
        for li in range(3):
            f0, f1, f2 = wrefs[3 * li : 3 * li + 3]
            h0 = _silu(jnp.dot(emb, f0[...], preferred_element_type=jnp.float32) / np.sqrt(NUM_BASIS))
            h1 = _silu(jnp.dot(h0, f1[...], preferred_element_type=jnp.float32) / np.sqrt(128.0))
            for hh in range(hs[li]):
                w = jnp.dot(h1, f2[hh], preferred_element_type=jnp.float32) / np.sqrt(128.0)
                if hs[li] == 1:
                    orefs[li][...] = w * y
                else:
                    orefs[li][hh] = w * y

    in_specs = [pl.BlockSpec((_BE, 1), lambda i: (i, 0))]
    args = [d2]
    for w in layers:
        args += [w["fc_w0"], w["fc_w1"], w["fc_w2h"]]
        in_specs += [
            pl.BlockSpec((NUM_BASIS, 128), lambda i: (0, 0)),
            pl.BlockSpec((128, 128), lambda i: (0, 0)),
            pl.BlockSpec((w["fc_w2h"].shape[0], 128, 128), lambda i: (0, 0, 0)),
        ]
    out_shapes = []
    out_specs = []
    for h in hs:
        if h == 1:
            out_shapes.append(jax.ShapeDtypeStruct((EP, 128), jnp.float32))
            out_specs.append(pl.BlockSpec((_BE, 128), lambda i: (i, 0)))
        else:
            out_shapes.append(jax.ShapeDtypeStruct((h, EP, 128), jnp.float32))
            out_specs.append(pl.BlockSpec((h, _BE, 128), lambda i: (0, i, 0)))
    outs = pl.pallas_call(
        body,
        grid=(EP // _BE,),
        in_specs=in_specs,
        out_specs=out_specs,
        out_shape=out_shapes,
    )(*args)
    return [o.reshape(-1, 128) for o in outs]


def _pre(h, wsc, w1h, h2=None, p=None):
    d_in = h.shape[1]
    d_out = wsc.shape[1]
    H = w1h.shape[0]

    def body(*refs):
        if p is None:
            h_ref, wsc_ref, w1_ref, xs_ref, xl_ref = refs
            hb = h_ref[...]
        else:
            h_ref, h2_ref, p_ref, wsc_ref, w1_ref, xs_ref, xl_ref = refs
            pv = p_ref[0, 0]
            hb = pv * h_ref[...] + (1.0 - pv) * h2_ref[...]
        c_s = np.float32(math.sin(math.pi / 8))
        xs_ref[...] = c_s * (
            jnp.dot(hb, wsc_ref[...], preferred_element_type=jnp.float32) / np.sqrt(d_in)
        )
        for k in range(H):
            xl_ref[k] = (
                jnp.dot(hb, w1_ref[k], preferred_element_type=jnp.float32) / np.sqrt(d_in)
            )

    in_specs = [pl.BlockSpec((_BN, d_in), lambda i: (i, 0))]
    args = [h]
    if p is not None:
        in_specs += [
            pl.BlockSpec((_BN, d_in), lambda i: (i, 0)),
            pl.BlockSpec((1, 1), lambda i: (0, 0)),
        ]
        args += [h2, p.reshape(1, 1)]
    in_specs += [
        pl.BlockSpec((d_in, d_out), lambda i: (0, 0)),
        pl.BlockSpec((H, d_in, 128), lambda i: (0, 0, 0)),
    ]
    args += [wsc, w1h]
    xs, xl = pl.pallas_call(
        body,
        grid=(N // _BN,),
        in_specs=in_specs,
        out_specs=[
            pl.BlockSpec((_BN, d_out), lambda i: (i, 0)),
            pl.BlockSpec((H, _BN, 128), lambda i: (0, i, 0)),
        ],
        out_shape=[
            jax.ShapeDtypeStruct((N, d_out), jnp.float32),
            jax.ShapeDtypeStruct((H, N, 128), jnp.float32),
        ],
    )(*args)
    return xs, xl.reshape(H * N, 128)


def _combine(agg, xs, w2h, d_in, act):
    d_out = xs.shape[1]
    H = w2h.shape[0]
    a = agg.reshape(2, _NP, 128)

    def body(a_ref, xs_ref, w2_ref, o_ref):
        c_x = np.float32(math.cos(math.pi / 8))
        if H == 1:
            ag = (a_ref[0] + a_ref[1]) / np.sqrt(32.0)
            out = jnp.dot(ag, w2_ref[0], preferred_element_type=jnp.float32)
        else:
            out = jnp.dot(
                a_ref[0] / np.sqrt(32.0), w2_ref[0], preferred_element_type=jnp.float32
            ) + jnp.dot(
                a_ref[1] / np.sqrt(32.0), w2_ref[1], preferred_element_type=jnp.float32
            )
        o = xs_ref[...] + c_x * (out / np.sqrt(d_in))
        if act:
            o = _silu(o)
        o_ref[...] = o

    return pl.pallas_call(
        body,
        grid=(N // _BN,),
        in_specs=[
            pl.BlockSpec((2, _BN, 128), lambda i: (0, i, 0)),
            pl.BlockSpec((_BN, d_out), lambda i: (i, 0)),
            pl.BlockSpec((H, 128, d_out), lambda i: (0, 0, 0)),
        ],
        out_specs=pl.BlockSpec((_BN, d_out), lambda i: (i, 0)),
        out_shape=jax.ShapeDtypeStruct((N, d_out), jnp.float32),
    )(a, xs, w2h)


def _prep_params(layers):
    out = []
    for lp in layers:
        d_in = lp["W_sc"].shape[0]
        H = d_in // 128
        w1 = lp["W_lin1"].reshape(d_in, H, 128)
        f2 = lp["fc_w2"].reshape(128, H, 128)
        out.append(
            {
                "W_sc": lp["W_sc"],
                "W_lin1h": jnp.transpose(w1, (1, 0, 2)),
                "fc_w0": lp["fc_w0"],
                "fc_w1": lp["fc_w1"],
                "fc_w2h": jnp.transpose(f2, (1, 0, 2)),
                "W_lin2h": lp["W_lin2"].reshape(H, 128, -1),
                "H": H,
                "d_in": d_in,
            }
        )
    return out


def _network(h, pos, src, dst, layers, h2=None, p=None):
    px, py, pz = pos[:, 0], pos[:, 1], pos[:, 2]
    dl2 = _geom(px, py, pz, src, dst)
    wes = _radial(dl2, layers)
    for li, lp in enumerate(layers):
        if li == 0 and p is not None:
            xs, xl = _pre(h, lp["W_sc"], lp["W_lin1h"], h2=h2, p=p)
        else:
            xs, xl = _pre(h, lp["W_sc"], lp["W_lin1h"])
        agg = _sparse_layer(lp["H"] == 2, xl, wes[li], src, dst)
        h = _combine(agg, xs, lp["W_lin2h"], lp["d_in"], act=(li < 2))
    return h


def kernel(x, x_final_state, pos, pos_final_state, pos_interpolated_transition_state, p, edge_index, batch, params):
    src = edge_index[0].astype(jnp.int32)
    dst = edge_index[1].astype(jnp.int32)
    net_i = _prep_params(params["net_init"])
    net_f = _prep_params(params["net_final"])
    net_ts = _prep_params(params["net_ts"])
    out_i = _network(x, pos, src, dst, net_i)
    out_f = _network(x_final_state, pos_final_state, src, dst, net_f)
    out_ts = _network(
        out_i,
        pos_interpolated_transition_state,
        src,
        dst,
        net_ts,
        h2=out_f,
        p=p[0],
    )
    return out_ts


# speedup vs baseline: 2.0227x; 1.1627x over previous
import dataclasses
import functools
import math

import jax
import jax.numpy as jnp
import numpy as np
from jax import lax
from jax.experimental import pallas as pl
from jax.experimental.pallas import tpu as pltpu
from jax.experimental.pallas import tpu_sc as plsc

N = 10000
E = 320000
EP = 327680
NUM_BASIS = 10
MAX_RADIUS = 5.0

_MESH = plsc.VectorSubcoreMesh(core_axis_name="c", subcore_axis_name="s")
_SC_PARAMS = pltpu.CompilerParams()
if "needs_layout_passes" in pltpu.CompilerParams.__dataclass_fields__:
    _SC_PARAMS = dataclasses.replace(_SC_PARAMS, needs_layout_passes=False)
_NS = 16
_EPT_G = E // 32
_ROWS_PT = 632
_NP = _ROWS_PT * _NS
_CHUNK = 80

_BE = 2048
_BN = 2000


def _silu(v):
    return v * jax.nn.sigmoid(v)


def _geom(px, py, pz, src, dst):
    @functools.partial(
        pl.kernel,
        out_type=jax.ShapeDtypeStruct((EP,), jnp.float32),
        mesh=_MESH,
        compiler_params=_SC_PARAMS,
        scratch_types=[
            pltpu.VMEM((N,), jnp.float32),
            pltpu.VMEM((N,), jnp.float32),
            pltpu.VMEM((N,), jnp.float32),
            pltpu.VMEM((_EPT_G,), jnp.int32),
            pltpu.VMEM((_EPT_G,), jnp.int32),
            pltpu.VMEM((_EPT_G,), jnp.float32),
        ],
    )
    def k(px_h, py_h, pz_h, src_h, dst_h, out_h, px_v, py_v, pz_v, si_v, di_v, o_v):
        c = lax.axis_index("c")
        s = lax.axis_index("s")
        base = (c * _NS + s) * _EPT_G
        pltpu.sync_copy(px_h, px_v)
        pltpu.sync_copy(py_h, py_v)
        pltpu.sync_copy(pz_h, pz_v)
        pltpu.sync_copy(src_h.at[pl.ds(base, _EPT_G)], si_v)
        pltpu.sync_copy(dst_h.at[pl.ds(base, _EPT_G)], di_v)

        @pl.loop(0, _EPT_G, step=16)
        def _(i):
            sl = pl.ds(i, 16)
            a = si_v[sl]
            b = di_v[sl]
            dx = plsc.load_gather(px_v, [a]) - plsc.load_gather(px_v, [b])
            dy = plsc.load_gather(py_v, [a]) - plsc.load_gather(py_v, [b])
            dz = plsc.load_gather(pz_v, [a]) - plsc.load_gather(pz_v, [b])
            o_v[sl] = dx * dx + dy * dy + dz * dz

        pltpu.sync_copy(o_v, out_h.at[pl.ds(base, _EPT_G)])

    return k(px, py, pz, src, dst)


_NR = EP // 128


def _sparse_layer(split, xl, we, src2, dst2):
    nchunks = (_NR * 2) // _NS if split else _NR // _NS
    nblocks = nchunks // 32

    @functools.partial(
        pl.kernel,
        out_type=jax.ShapeDtypeStruct((2 * _NP, 128), jnp.float32),
        mesh=_MESH,
        compiler_params=_SC_PARAMS,
        scratch_types=[
            pltpu.VMEM((32, 64), jnp.int32),
            pltpu.VMEM((32, 64), jnp.int32),
            pltpu.VMEM((64, 128), jnp.float32),
            pltpu.VMEM((64, 128), jnp.float32),
            pltpu.VMEM((64, 128), jnp.float32),
            pltpu.VMEM((64, 128), jnp.float32),
            pltpu.SemaphoreType.DMA,
            pltpu.SemaphoreType.DMA,
            pltpu.SemaphoreType.DMA,
            pltpu.SemaphoreType.DMA,
            pltpu.VMEM_SHARED((_NP, 128), jnp.float32),
        ],
    )
    def k(xl_h, we_h, src_h, dst_h, out_h, si_b, di_b, r0, r1, e0, e1,
          g0, g1, w0, w1, agg_sh):
        c = lax.axis_index("c")
        s = lax.axis_index("s")
        cbase = (s * nchunks) if split else ((c * _NS + s) * nchunks)
        ebase = cbase * 64

        @pl.loop(0, 64)
        def _(i):
            for jj in range(8):
                r0[i, pl.ds(jj * 16, 16)] = jnp.zeros((16,), jnp.float32)

        zbase = s * _ROWS_PT
        for t in range(9):
            pltpu.sync_copy(r0, agg_sh.at[pl.ds(zbase + t * 64, 64)])
        pltpu.sync_copy(r0.at[pl.ds(0, 56)], agg_sh.at[pl.ds(zbase + 576, 56)])
        plsc.subcore_barrier()

        web = (c * EP if split else 0) + ebase
        off = c * N

        @pl.loop(0, nblocks)
        def _(b):
            brow = cbase + b * 32
            pltpu.sync_copy(src_h.at[pl.ds(brow, 32)], si_b)
            pltpu.sync_copy(dst_h.at[pl.ds(brow, 32)], di_b)
            if split:

                @pl.loop(0, 32)
                def _(i):
                    for jj in range(4):
                        sl = pl.ds(jj * 16, 16)
                        si_b[i, sl] = si_b[i, sl] + off

            eb = web + b * 2048

            def gcopy(buf, sem, cc):
                return pltpu.make_async_copy(xl_h.at[si_b.at[cc]], buf, sem)

            def wcopy(buf, sem, cc):
                return pltpu.make_async_copy(
                    we_h.at[pl.ds(eb + cc * 64, 64)], buf, sem
                )

            gcopy(r0, g0, 0).start()
            wcopy(e0, w0, 0).start()
            gcopy(r1, g1, 1).start()
            wcopy(e1, w1, 1).start()

            def slot(rv, ev, gs, ws, tt, nxt):
                gcopy(rv, gs, tt).wait()
                wcopy(ev, ws, tt).wait()

                @pl.loop(0, 64, step=2)
                def _(i):
                    for ii in range(2):
                        for jj in range(8):
                            sl = pl.ds(jj * 16, 16)
                            rv[i + ii, sl] = rv[i + ii, sl] * ev[i + ii, sl]

                pltpu.sync_copy(rv, agg_sh.at[di_b.at[tt]], add=True)

                @pl.when(nxt < 32)
                def _():
                    gcopy(rv, gs, nxt).start()
                    wcopy(ev, ws, nxt).start()

            @pl.loop(0, 32, step=2)
            def _(t):
                slot(r0, e0, g0, w0, t, t + 2)
                slot(r1, e1, g1, w1, t + 1, t + 3)

        plsc.subcore_barrier()
        pltpu.sync_copy(
            agg_sh.at[pl.ds(zbase, _ROWS_PT)],
            out_h.at[pl.ds(c * _NP + zbase, _ROWS_PT)],
        )

    return k(xl, we, src2, dst2)


def _radial(dl2, layers):
    d2 = dl2.reshape(EP, 1)
    vals = np.linspace(0.0, MAX_RADIUS, NUM_BASIS).astype(np.float32)
    step = float(vals[1] - vals[0])
    hs = [w["fc_w2h"].shape[0] for w in layers]

    def body(d2_ref, *refs):
        wrefs = refs[:9]
        orefs = refs[9:]
        el = jnp.sqrt(d2_ref[...] + 1e-9)
        vgrid = (
            lax.broadcasted_iota(jnp.int32, (1, NUM_BASIS), 1).astype(jnp.float32)
            * step
        )
        diff = (el - vgrid) / step
        emb = (jnp.exp(-diff * diff) / 1.12) * (NUM_BASIS ** 0.5)
        u = 2.0 * (el / MAX_RADIUS - 1.0)
        y = (1.0 - jnp.cos(np.float32(math.pi) * u)) / 2.0
        y = jnp.where(u > 0.0, 0.0, y)
        y = jnp.where(u < -1.0, 1.0, y)
        for li in range(3):
            f0, f1, f2 = wrefs[3 * li : 3 * li + 3]
            h0 = _silu(jnp.dot(emb, f0[...], preferred_element_type=jnp.float32) / np.sqrt(NUM_BASIS))
            h1 = _silu(jnp.dot(h0, f1[...], preferred_element_type=jnp.float32) / np.sqrt(128.0))
            for hh in range(hs[li]):
                w = jnp.dot(h1, f2[hh], preferred_element_type=jnp.float32) / np.sqrt(128.0)
                if hs[li] == 1:
                    orefs[li][...] = w * y
                else:
                    orefs[li][hh] = w * y

    in_specs = [pl.BlockSpec((_BE, 1), lambda i: (i, 0))]
    args = [d2]
    for w in layers:
        args += [w["fc_w0"], w["fc_w1"], w["fc_w2h"]]
        in_specs += [
            pl.BlockSpec((NUM_BASIS, 128), lambda i: (0, 0)),
            pl.BlockSpec((128, 128), lambda i: (0, 0)),
            pl.BlockSpec((w["fc_w2h"].shape[0], 128, 128), lambda i: (0, 0, 0)),
        ]
    out_shapes = []
    out_specs = []
    for h in hs:
        if h == 1:
            out_shapes.append(jax.ShapeDtypeStruct((EP, 128), jnp.float32))
            out_specs.append(pl.BlockSpec((_BE, 128), lambda i: (i, 0)))
        else:
            out_shapes.append(jax.ShapeDtypeStruct((h, EP, 128), jnp.float32))
            out_specs.append(pl.BlockSpec((h, _BE, 128), lambda i: (0, i, 0)))
    outs = pl.pallas_call(
        body,
        grid=(EP // _BE,),
        in_specs=in_specs,
        out_specs=out_specs,
        out_shape=out_shapes,
    )(*args)
    return [o.reshape(-1, 128) for o in outs]


def _pre(h, wsc, w1h, h2=None, p=None):
    d_in = h.shape[1]
    d_out = wsc.shape[1]
    H = w1h.shape[0]

    def body(*refs):
        if p is None:
            h_ref, wsc_ref, w1_ref, xs_ref, xl_ref = refs
            hb = h_ref[...]
        else:
            h_ref, h2_ref, p_ref, wsc_ref, w1_ref, xs_ref, xl_ref = refs
            pv = p_ref[0, 0]
            hb = pv * h_ref[...] + (1.0 - pv) * h2_ref[...]
        c_s = np.float32(math.sin(math.pi / 8))
        xs_ref[...] = c_s * (
            jnp.dot(hb, wsc_ref[...], preferred_element_type=jnp.float32) / np.sqrt(d_in)
        )
        for k in range(H):
            xl_ref[k] = (
                jnp.dot(hb, w1_ref[k], preferred_element_type=jnp.float32) / np.sqrt(d_in)
            )

    in_specs = [pl.BlockSpec((_BN, d_in), lambda i: (i, 0))]
    args = [h]
    if p is not None:
        in_specs += [
            pl.BlockSpec((_BN, d_in), lambda i: (i, 0)),
            pl.BlockSpec((1, 1), lambda i: (0, 0)),
        ]
        args += [h2, p.reshape(1, 1)]
    in_specs += [
        pl.BlockSpec((d_in, d_out), lambda i: (0, 0)),
        pl.BlockSpec((H, d_in, 128), lambda i: (0, 0, 0)),
    ]
    args += [wsc, w1h]
    xs, xl = pl.pallas_call(
        body,
        grid=(N // _BN,),
        in_specs=in_specs,
        out_specs=[
            pl.BlockSpec((_BN, d_out), lambda i: (i, 0)),
            pl.BlockSpec((H, _BN, 128), lambda i: (0, i, 0)),
        ],
        out_shape=[
            jax.ShapeDtypeStruct((N, d_out), jnp.float32),
            jax.ShapeDtypeStruct((H, N, 128), jnp.float32),
        ],
    )(*args)
    return xs, xl.reshape(H * N, 128)


def _combine(agg, xs, w2h, d_in, act):
    d_out = xs.shape[1]
    H = w2h.shape[0]
    a = agg.reshape(2, _NP, 128)

    def body(a_ref, xs_ref, w2_ref, o_ref):
        c_x = np.float32(math.cos(math.pi / 8))
        if H == 1:
            ag = (a_ref[0] + a_ref[1]) / np.sqrt(32.0)
            out = jnp.dot(ag, w2_ref[0], preferred_element_type=jnp.float32)
        else:
            out = jnp.dot(
                a_ref[0] / np.sqrt(32.0), w2_ref[0], preferred_element_type=jnp.float32
            ) + jnp.dot(
                a_ref[1] / np.sqrt(32.0), w2_ref[1], preferred_element_type=jnp.float32
            )
        o = xs_ref[...] + c_x * (out / np.sqrt(d_in))
        if act:
            o = _silu(o)
        o_ref[...] = o

    return pl.pallas_call(
        body,
        grid=(N // _BN,),
        in_specs=[
            pl.BlockSpec((2, _BN, 128), lambda i: (0, i, 0)),
            pl.BlockSpec((_BN, d_out), lambda i: (i, 0)),
            pl.BlockSpec((H, 128, d_out), lambda i: (0, 0, 0)),
        ],
        out_specs=pl.BlockSpec((_BN, d_out), lambda i: (i, 0)),
        out_shape=jax.ShapeDtypeStruct((N, d_out), jnp.float32),
    )(a, xs, w2h)


def _prep_params(layers):
    out = []
    for lp in layers:
        d_in = lp["W_sc"].shape[0]
        H = d_in // 128
        w1 = lp["W_lin1"].reshape(d_in, H, 128)
        f2 = lp["fc_w2"].reshape(128, H, 128)
        out.append(
            {
                "W_sc": lp["W_sc"],
                "W_lin1h": jnp.transpose(w1, (1, 0, 2)),
                "fc_w0": lp["fc_w0"],
                "fc_w1": lp["fc_w1"],
                "fc_w2h": jnp.transpose(f2, (1, 0, 2)),
                "W_lin2h": lp["W_lin2"].reshape(H, 128, -1),
                "H": H,
                "d_in": d_in,
            }
        )
    return out


def _network(h, pos, src, dst, src2, dst2, layers, h2=None, p=None):
    px, py, pz = pos[:, 0], pos[:, 1], pos[:, 2]
    dl2 = _geom(px, py, pz, src, dst)
    wes = _radial(dl2, layers)
    for li, lp in enumerate(layers):
        if li == 0 and p is not None:
            xs, xl = _pre(h, lp["W_sc"], lp["W_lin1h"], h2=h2, p=p)
        else:
            xs, xl = _pre(h, lp["W_sc"], lp["W_lin1h"])
        agg = _sparse_layer(lp["H"] == 2, xl, wes[li], src2, dst2)
        h = _combine(agg, xs, lp["W_lin2h"], lp["d_in"], act=(li < 2))
    return h


def kernel(x, x_final_state, pos, pos_final_state, pos_interpolated_transition_state, p, edge_index, batch, params):
    src = edge_index[0].astype(jnp.int32)
    dst = edge_index[1].astype(jnp.int32)
    src2 = jnp.concatenate([src, jnp.zeros((EP - E,), jnp.int32)]).reshape(EP // 64, 64)
    dst2 = jnp.concatenate([dst, jnp.full((EP - E,), N, jnp.int32)]).reshape(EP // 64, 64)
    net_i = _prep_params(params["net_init"])
    net_f = _prep_params(params["net_final"])
    net_ts = _prep_params(params["net_ts"])
    out_i = _network(x, pos, src, dst, src2, dst2, net_i)
    out_f = _network(x_final_state, pos_final_state, src, dst, src2, dst2, net_f)
    out_ts = _network(
        out_i,
        pos_interpolated_transition_state,
        src,
        dst,
        src2,
        dst2,
        net_ts,
        h2=out_f,
        p=p[0],
    )
    return out_ts
